# Initial kernel scaffold; baseline (speedup 1.0000x reference)
import jax
import jax.numpy as jnp
from jax.experimental import pallas as pl


def kernel(x, w1, b1, w2, b2, w3, b3):
    raise NotImplementedError("write your pallas kernel here")


# speedup vs baseline: 1.7401x; 1.7401x over previous
import jax
import jax.numpy as jnp
from jax.experimental import pallas as pl
from jax.experimental.pallas import tpu as pltpu

_LANE = 128
_SUBLANE = 8
_TILE_B = 1024


def _rup(n, m):
    return (n + m - 1) // m * m


def _pad2(a, rows, cols):
    if a.shape == (rows, cols):
        return a
    return jnp.pad(a, ((0, rows - a.shape[0]), (0, cols - a.shape[1])))


def _mlp_kernel(x_ref, w1_ref, b1_ref, w2_ref, b2_ref, w3_ref, b3_ref, o_ref):
    x = x_ref[...].astype(jnp.bfloat16)
    h1 = jnp.dot(x, w1_ref[...], preferred_element_type=jnp.float32)
    h1 = jnp.maximum(h1 + b1_ref[...], 0.0).astype(jnp.bfloat16)
    h2 = jnp.dot(h1, w2_ref[...], preferred_element_type=jnp.float32)
    h2 = jnp.maximum(h2 + b2_ref[...], 0.0).astype(jnp.bfloat16)
    out = jnp.dot(h2, w3_ref[...], preferred_element_type=jnp.float32)
    o_ref[...] = out + b3_ref[...]


@jax.jit
def kernel(x, w1, b1, w2, b2, w3, b3):
    B, in_size = x.shape
    hidden_size = w1.shape[1]
    out_size = w3.shape[1]

    in_p = _rup(in_size, _LANE)
    hid_p = _rup(hidden_size, _LANE)
    out_p = _rup(out_size, _LANE)

    tile_b = min(_TILE_B, _rup(B, _SUBLANE))
    b_pad = _rup(B, tile_b)

    x_p = _pad2(x.astype(jnp.float32), b_pad, in_p)
    w1_p = _pad2(w1, in_p, hid_p).astype(jnp.bfloat16)
    w2_p = _pad2(w2, hid_p, hid_p).astype(jnp.bfloat16)
    w3_p = _pad2(w3, hid_p, out_p).astype(jnp.bfloat16)
    b1_p = _pad2(b1, 1, hid_p)
    b2_p = _pad2(b2, 1, hid_p)
    b3_p = _pad2(b3, 1, out_p)

    const = lambda i: (0, 0)
    out_padded = pl.pallas_call(
        _mlp_kernel,
        out_shape=jax.ShapeDtypeStruct((b_pad, out_p), jnp.float32),
        grid=(b_pad // tile_b,),
        in_specs=[
            pl.BlockSpec((tile_b, in_p), lambda i: (i, 0)),
            pl.BlockSpec((in_p, hid_p), const),
            pl.BlockSpec((1, hid_p), const),
            pl.BlockSpec((hid_p, hid_p), const),
            pl.BlockSpec((1, hid_p), const),
            pl.BlockSpec((hid_p, out_p), const),
            pl.BlockSpec((1, out_p), const),
        ],
        out_specs=pl.BlockSpec((tile_b, out_p), lambda i: (i, 0)),
        compiler_params=pltpu.CompilerParams(
            dimension_semantics=("parallel",)),
        name="qnet_mlp_bf16",
    )(x_p, w1_p, b1_p, w2_p, b2_p, w3_p, b3_p)

    if (b_pad, out_p) != (B, out_size):
        out_padded = out_padded[:B, :out_size]
    return out_padded


# speedup vs baseline: 1.9024x; 1.0933x over previous
import jax
import jax.numpy as jnp
from jax.experimental import pallas as pl
from jax.experimental.pallas import tpu as pltpu

_LANE = 128
_SUBLANE = 8
_TILE_B = 2048


def _rup(n, m):
    return (n + m - 1) // m * m


def _pad2(a, rows, cols):
    if a.shape == (rows, cols):
        return a
    return jnp.pad(a, ((0, rows - a.shape[0]), (0, cols - a.shape[1])))


def _mlp_kernel(x_ref, w1_ref, b1_ref, w2_ref, b2_ref, w3_ref, b3_ref, o_ref):
    x = x_ref[...].astype(jnp.bfloat16)
    h1 = jnp.dot(x, w1_ref[...], preferred_element_type=jnp.float32)
    h1 = jnp.maximum(h1 + b1_ref[...], 0.0).astype(jnp.bfloat16)
    h2 = jnp.dot(h1, w2_ref[...], preferred_element_type=jnp.float32)
    h2 = jnp.maximum(h2 + b2_ref[...], 0.0).astype(jnp.bfloat16)
    out = jnp.dot(h2, w3_ref[...], preferred_element_type=jnp.float32)
    o_ref[...] = out + b3_ref[...]


@jax.jit
def kernel(x, w1, b1, w2, b2, w3, b3):
    B, in_size = x.shape
    hidden_size = w1.shape[1]
    out_size = w3.shape[1]

    in_p = _rup(in_size, _LANE)
    hid_p = _rup(hidden_size, _LANE)
    out_p = _rup(out_size, _LANE)

    tile_b = min(_TILE_B, _rup(B, _SUBLANE))
    b_pad = _rup(B, tile_b)

    x_p = _pad2(x.astype(jnp.float32), b_pad, in_p)
    w1_p = _pad2(w1, in_p, hid_p).astype(jnp.bfloat16)
    w2_p = _pad2(w2, hid_p, hid_p).astype(jnp.bfloat16)
    w3_p = _pad2(w3, hid_p, out_p).astype(jnp.bfloat16)
    b1_p = _pad2(b1, 1, hid_p)
    b2_p = _pad2(b2, 1, hid_p)
    b3_p = _pad2(b3, 1, out_p)

    const = lambda i: (0, 0)
    out_padded = pl.pallas_call(
        _mlp_kernel,
        out_shape=jax.ShapeDtypeStruct((b_pad, out_p), jnp.float32),
        grid=(b_pad // tile_b,),
        in_specs=[
            pl.BlockSpec((tile_b, in_p), lambda i: (i, 0)),
            pl.BlockSpec((in_p, hid_p), const),
            pl.BlockSpec((1, hid_p), const),
            pl.BlockSpec((hid_p, hid_p), const),
            pl.BlockSpec((1, hid_p), const),
            pl.BlockSpec((hid_p, out_p), const),
            pl.BlockSpec((1, out_p), const),
        ],
        out_specs=pl.BlockSpec((tile_b, out_p), lambda i: (i, 0)),
        compiler_params=pltpu.CompilerParams(
            dimension_semantics=("parallel",)),
        name="qnet_mlp_bf16",
    )(x_p, w1_p, b1_p, w2_p, b2_p, w3_p, b3_p)

    if (b_pad, out_p) != (B, out_size):
        out_padded = out_padded[:B, :out_size]
    return out_padded


# speedup vs baseline: 1.9368x; 1.0180x over previous
import jax
import jax.numpy as jnp
from jax.experimental import pallas as pl
from jax.experimental.pallas import tpu as pltpu

_LANE = 128
_SUBLANE = 8
_TILE_B = 4096


def _rup(n, m):
    return (n + m - 1) // m * m


def _pad2(a, rows, cols):
    if a.shape == (rows, cols):
        return a
    return jnp.pad(a, ((0, rows - a.shape[0]), (0, cols - a.shape[1])))


def _mlp_kernel(x_ref, w1_ref, b1_ref, w2_ref, b2_ref, w3_ref, b3_ref, o_ref):
    x = x_ref[...].astype(jnp.bfloat16)
    h1 = jnp.dot(x, w1_ref[...], preferred_element_type=jnp.float32)
    h1 = jnp.maximum(h1 + b1_ref[...], 0.0).astype(jnp.bfloat16)
    h2 = jnp.dot(h1, w2_ref[...], preferred_element_type=jnp.float32)
    h2 = jnp.maximum(h2 + b2_ref[...], 0.0).astype(jnp.bfloat16)
    out = jnp.dot(h2, w3_ref[...], preferred_element_type=jnp.float32)
    o_ref[...] = out + b3_ref[...]


@jax.jit
def kernel(x, w1, b1, w2, b2, w3, b3):
    B, in_size = x.shape
    hidden_size = w1.shape[1]
    out_size = w3.shape[1]

    in_p = _rup(in_size, _LANE)
    hid_p = _rup(hidden_size, _LANE)
    out_p = _rup(out_size, _LANE)

    tile_b = min(_TILE_B, _rup(B, _SUBLANE))
    b_pad = _rup(B, tile_b)

    x_p = _pad2(x.astype(jnp.float32), b_pad, in_p)
    w1_p = _pad2(w1, in_p, hid_p).astype(jnp.bfloat16)
    w2_p = _pad2(w2, hid_p, hid_p).astype(jnp.bfloat16)
    w3_p = _pad2(w3, hid_p, out_p).astype(jnp.bfloat16)
    b1_p = _pad2(b1, 1, hid_p)
    b2_p = _pad2(b2, 1, hid_p)
    b3_p = _pad2(b3, 1, out_p)

    const = lambda i: (0, 0)
    out_padded = pl.pallas_call(
        _mlp_kernel,
        out_shape=jax.ShapeDtypeStruct((b_pad, out_p), jnp.float32),
        grid=(b_pad // tile_b,),
        in_specs=[
            pl.BlockSpec((tile_b, in_p), lambda i: (i, 0)),
            pl.BlockSpec((in_p, hid_p), const),
            pl.BlockSpec((1, hid_p), const),
            pl.BlockSpec((hid_p, hid_p), const),
            pl.BlockSpec((1, hid_p), const),
            pl.BlockSpec((hid_p, out_p), const),
            pl.BlockSpec((1, out_p), const),
        ],
        out_specs=pl.BlockSpec((tile_b, out_p), lambda i: (i, 0)),
        compiler_params=pltpu.CompilerParams(
            dimension_semantics=("parallel",)),
        name="qnet_mlp_bf16",
    )(x_p, w1_p, b1_p, w2_p, b2_p, w3_p, b3_p)

    if (b_pad, out_p) != (B, out_size):
        out_padded = out_padded[:B, :out_size]
    return out_padded


# speedup vs baseline: 2.3913x; 1.2347x over previous
import jax
import jax.numpy as jnp
from jax.experimental import pallas as pl
from jax.experimental.pallas import tpu as pltpu

_LANE = 128
_SUBLANE = 8
_TILE_B = 4096


def _rup(n, m):
    return (n + m - 1) // m * m


def _pad2(a, rows, cols):
    if a.shape == (rows, cols):
        return a
    return jnp.pad(a, ((0, rows - a.shape[0]), (0, cols - a.shape[1])))


def _mlp_kernel(x_ref, w1_ref, b1_ref, w2_ref, b2_ref, w3_ref, b3_ref, o_ref):
    x = x_ref[...].astype(jnp.bfloat16)
    w1 = w1_ref[...].astype(jnp.bfloat16)
    w2 = w2_ref[...].astype(jnp.bfloat16)
    w3 = w3_ref[...].astype(jnp.bfloat16)
    h1 = jnp.dot(x, w1, preferred_element_type=jnp.float32)
    h1 = jnp.maximum(h1 + b1_ref[...], 0.0).astype(jnp.bfloat16)
    h2 = jnp.dot(h1, w2, preferred_element_type=jnp.float32)
    h2 = jnp.maximum(h2 + b2_ref[...], 0.0).astype(jnp.bfloat16)
    out = jnp.dot(h2, w3, preferred_element_type=jnp.float32)
    o_ref[...] = out + b3_ref[...]


@jax.jit
def kernel(x, w1, b1, w2, b2, w3, b3):
    B, in_size = x.shape
    hidden_size = w1.shape[1]
    out_size = w3.shape[1]

    in_p = _rup(in_size, _LANE)
    hid_p = _rup(hidden_size, _LANE)
    out_p = _rup(out_size, _LANE)

    tile_b = min(_TILE_B, _rup(B, _SUBLANE))
    b_pad = _rup(B, tile_b)

    x_p = _pad2(x, b_pad, in_p)
    w1_p = _pad2(w1, in_p, hid_p)
    w2_p = _pad2(w2, hid_p, hid_p)
    w3_p = _pad2(w3, hid_p, out_p)
    b1_p = _pad2(b1, 1, hid_p)
    b2_p = _pad2(b2, 1, hid_p)
    b3_p = _pad2(b3, 1, out_p)

    const = lambda i: (0, 0)
    out_padded = pl.pallas_call(
        _mlp_kernel,
        out_shape=jax.ShapeDtypeStruct((b_pad, out_p), jnp.float32),
        grid=(b_pad // tile_b,),
        in_specs=[
            pl.BlockSpec((tile_b, in_p), lambda i: (i, 0)),
            pl.BlockSpec((in_p, hid_p), const),
            pl.BlockSpec((1, hid_p), const),
            pl.BlockSpec((hid_p, hid_p), const),
            pl.BlockSpec((1, hid_p), const),
            pl.BlockSpec((hid_p, out_p), const),
            pl.BlockSpec((1, out_p), const),
        ],
        out_specs=pl.BlockSpec((tile_b, out_p), lambda i: (i, 0)),
        compiler_params=pltpu.CompilerParams(
            dimension_semantics=("parallel",)),
        name="qnet_mlp_bf16",
    )(x_p, w1_p, b1_p, w2_p, b2_p, w3_p, b3_p)

    if (b_pad, out_p) != (B, out_size):
        out_padded = out_padded[:B, :out_size]
    return out_padded
